# baseline (speedup 1.0000x reference)
import functools

import jax
import jax.numpy as jnp
from jax import lax
from jax.experimental import pallas as pl
from jax.experimental.pallas import tpu as pltpu
from jax.experimental.pallas import tpu_sc as plsc

N_REAL = 5000
N_PAD = 5120
ROW_BLK = 1024
RB = N_PAD // ROW_BLK
NW = 32
CHUNK = (4 * N_PAD) // NW
PAD_COORD = 1.0e15
FS_T = 0.1
BIG_I = 2 ** 30


def _norms_body(p_ref, g_ref, nha_ref, nb_ref):
    p = p_ref[0]
    g = g_ref[0]
    nha_ref[0] = (-0.5) * jnp.sum(p * p, axis=1, keepdims=True)
    nb_ref[0] = (-0.5) * jnp.sum(g * g, axis=0, keepdims=True)


def _chamfer_body(pb_ref, gb_ref, nha_ref, nb_ref,
                  ip_ref, cn_ref, ci_ref):
    rb = pl.program_id(1)
    pb = pb_ref[0]
    gb = gb_ref[0]
    nha = nha_ref[0]
    nb = nb_ref[0]
    u = lax.dot_general(pb, gb, (((0,), (0,)), ((), ())),
                        preferred_element_type=jnp.float32)
    t1 = u + nb
    colidx = lax.broadcasted_iota(jnp.int32, (ROW_BLK, N_PAD), 1)
    rn = jnp.max(t1, axis=1, keepdims=True)
    ip_ref[0] = jnp.min(jnp.where(t1 == rn, colidx, BIG_I),
                        axis=1, keepdims=True)
    del rb
    t2 = u + nha
    rowidx = lax.broadcasted_iota(jnp.int32, (ROW_BLK, N_PAD), 0)
    cn = jnp.max(t2, axis=0, keepdims=True)
    cn_ref[0, 0] = cn
    ci_ref[0, 0] = jnp.min(jnp.where(t2 == cn, rowidx, BIG_I), axis=0,
                           keepdims=True)


def _select_body(cn_ref, ci_ref, ig_ref):
    cn = cn_ref[...]
    ci = ci_ref[...]
    ci = ci + ROW_BLK * lax.broadcasted_iota(jnp.int32, ci.shape, 1)
    cg = jnp.max(cn, axis=1, keepdims=True)
    ig_ref[...] = jnp.min(jnp.where(cn == cg, ci, BIG_I), axis=1)


def _sc_exact(px, py, pz, gx, gy, gz, ip, ig, dp_out, dg_out,
              pxb, pyb, pzb, gxb, gyb, gzb, ipc, igc, dpc, dgc):
    nc = 2
    w = lax.axis_index("s") * nc + lax.axis_index("c")
    b = w // 8
    c0 = (w % 8) * CHUNK

    pltpu.sync_copy(px.at[pl.ds(b * N_PAD, N_PAD)], pxb)
    pltpu.sync_copy(py.at[pl.ds(b * N_PAD, N_PAD)], pyb)
    pltpu.sync_copy(pz.at[pl.ds(b * N_PAD, N_PAD)], pzb)
    pltpu.sync_copy(gx.at[pl.ds(b * N_PAD, N_PAD)], gxb)
    pltpu.sync_copy(gy.at[pl.ds(b * N_PAD, N_PAD)], gyb)
    pltpu.sync_copy(gz.at[pl.ds(b * N_PAD, N_PAD)], gzb)
    pltpu.sync_copy(ip.at[pl.ds(w * CHUNK, CHUNK)], ipc)
    pltpu.sync_copy(ig.at[pl.ds(w * CHUNK, CHUNK)], igc)

    def body(i, _):
        s = pl.ds(i * 16, 16)
        sc = pl.ds(c0 + i * 16, 16)
        iv = ipc[s]
        nx = plsc.load_gather(gxb, [iv])
        ny = plsc.load_gather(gyb, [iv])
        nz = plsc.load_gather(gzb, [iv])
        dx = pxb[sc] - nx
        dy = pyb[sc] - ny
        dz = pzb[sc] - nz
        dpc[s] = dx * dx + dy * dy + dz * dz
        jv = igc[s]
        mx = plsc.load_gather(pxb, [jv])
        my = plsc.load_gather(pyb, [jv])
        mz = plsc.load_gather(pzb, [jv])
        ex = gxb[sc] - mx
        ey = gyb[sc] - my
        ez = gzb[sc] - mz
        dgc[s] = ex * ex + ey * ey + ez * ez
        return 0

    lax.fori_loop(0, CHUNK // 16, body, 0, unroll=2)

    pltpu.sync_copy(dpc, dp_out.at[pl.ds(w * CHUNK, CHUNK)])
    pltpu.sync_copy(dgc, dg_out.at[pl.ds(w * CHUNK, CHUNK)])


def _finalize_body(dp_ref, dg_ref, out_ref):
    def stats(d2):
        mask = lax.broadcasted_iota(jnp.int32, d2.shape, 1) < N_REAL
        d = jnp.sqrt(jnp.maximum(d2, 1e-12))
        mean_d = jnp.sum(jnp.where(mask, d, 0.0), axis=1) / N_REAL
        frac = jnp.sum(jnp.where(mask & (d < FS_T), 1.0, 0.0), axis=1) / N_REAL
        return mean_d, frac

    mp, fp = stats(dp_ref[...])
    mg, fg = stats(dg_ref[...])
    cd = (mp + mg) / 2.0
    fs = 2.0 * fp * fg / (fp + fg + 1e-08)
    out_ref[...] = jnp.stack([cd, fs], axis=0)


def kernel(pred, gt):
    B, N, _ = pred.shape

    def prep(x):
        x = jnp.pad(x, ((0, 0), (0, N_PAD - N), (0, 0)),
                    constant_values=PAD_COORD)
        return jnp.pad(x, ((0, 0), (0, 0), (0, 5)))

    p8 = prep(pred)

    def prep_t(x):
        x = x.transpose(0, 2, 1)
        x = jnp.pad(x, ((0, 0), (0, 0), (0, N_PAD - N)),
                    constant_values=PAD_COORD)
        return jnp.pad(x, ((0, 0), (0, 5), (0, 0)))

    pt = prep_t(pred)
    gt8 = prep_t(gt)
    pbt = pt.astype(jnp.bfloat16)
    gbt = gt8.astype(jnp.bfloat16)

    nha, nb = pl.pallas_call(
        _norms_body,
        grid=(B,),
        in_specs=[
            pl.BlockSpec((1, N_PAD, 8), lambda b: (b, 0, 0)),
            pl.BlockSpec((1, 8, N_PAD), lambda b: (b, 0, 0)),
        ],
        out_specs=[
            pl.BlockSpec((1, N_PAD, 1), lambda b: (b, 0, 0)),
            pl.BlockSpec((1, 1, N_PAD), lambda b: (b, 0, 0)),
        ],
        out_shape=[
            jax.ShapeDtypeStruct((B, N_PAD, 1), jnp.float32),
            jax.ShapeDtypeStruct((B, 1, N_PAD), jnp.float32),
        ],
    )(p8, gt8)

    colspec = pl.BlockSpec((1, 1, 1, N_PAD), lambda b, rb: (b, rb, 0, 0))

    ip, cn, ci = pl.pallas_call(
        _chamfer_body,
        grid=(B, RB),
        in_specs=[
            pl.BlockSpec((1, 8, ROW_BLK), lambda b, rb: (b, 0, rb)),
            pl.BlockSpec((1, 8, N_PAD), lambda b, rb: (b, 0, 0)),
            pl.BlockSpec((1, ROW_BLK, 1), lambda b, rb: (b, rb, 0)),
            pl.BlockSpec((1, 1, N_PAD), lambda b, rb: (b, 0, 0)),
        ],
        out_specs=[
            pl.BlockSpec((1, ROW_BLK, 1), lambda b, rb: (b, rb, 0)),
            colspec, colspec,
        ],
        out_shape=[
            jax.ShapeDtypeStruct((B, N_PAD, 1), jnp.int32),
            jax.ShapeDtypeStruct((B, RB, 1, N_PAD), jnp.float32),
            jax.ShapeDtypeStruct((B, RB, 1, N_PAD), jnp.int32),
        ],
    )(pbt, gbt, nha, nb)

    ig = pl.pallas_call(
        _select_body,
        out_shape=jax.ShapeDtypeStruct((B, N_PAD), jnp.int32),
    )(cn.reshape(B, RB, N_PAD), ci.reshape(B, RB, N_PAD))

    p3 = pt[:, :3, :].transpose(1, 0, 2).reshape(3, B * N_PAD)
    g3 = gt8[:, :3, :].transpose(1, 0, 2).reshape(3, B * N_PAD)

    mesh = plsc.VectorSubcoreMesh(core_axis_name="c", subcore_axis_name="s",
                                  num_cores=2, num_subcores=16)
    sc = functools.partial(
        pl.kernel,
        mesh=mesh,
        compiler_params=pltpu.CompilerParams(needs_layout_passes=False),
        out_type=[
            jax.ShapeDtypeStruct((B * N_PAD,), jnp.float32),
            jax.ShapeDtypeStruct((B * N_PAD,), jnp.float32),
        ],
        scratch_types=[
            pltpu.VMEM((N_PAD,), jnp.float32),
            pltpu.VMEM((N_PAD,), jnp.float32),
            pltpu.VMEM((N_PAD,), jnp.float32),
            pltpu.VMEM((N_PAD,), jnp.float32),
            pltpu.VMEM((N_PAD,), jnp.float32),
            pltpu.VMEM((N_PAD,), jnp.float32),
            pltpu.VMEM((CHUNK,), jnp.int32),
            pltpu.VMEM((CHUNK,), jnp.int32),
            pltpu.VMEM((CHUNK,), jnp.float32),
            pltpu.VMEM((CHUNK,), jnp.float32),
        ],
    )(_sc_exact)
    dp, dg = sc(p3[0], p3[1], p3[2], g3[0], g3[1], g3[2],
                ip.reshape(B * N_PAD), ig.reshape(B * N_PAD))

    out = pl.pallas_call(
        _finalize_body,
        out_shape=jax.ShapeDtypeStruct((2, B), jnp.float32),
    )(dp.reshape(B, N_PAD), dg.reshape(B, N_PAD))
    return out


# speedup vs baseline: 1.0402x; 1.0402x over previous
import functools

import jax
import jax.numpy as jnp
from jax import lax
from jax.experimental import pallas as pl
from jax.experimental.pallas import tpu as pltpu
from jax.experimental.pallas import tpu_sc as plsc

N_REAL = 5000
N_PAD = 5120
ROW_BLK = 1024
RB = N_PAD // ROW_BLK
NW = 32
CHUNK = (4 * N_PAD) // NW
PAD_COORD = 1.0e15
FS_T = 0.1
BIG_I = 2 ** 30


def _norms_body(p_ref, g_ref, nha_ref, nb_ref):
    p = p_ref[0]
    g = g_ref[0]
    nha_ref[0] = (-0.5) * jnp.sum(p * p, axis=1, keepdims=True)
    nb_ref[0] = (-0.5) * jnp.sum(g * g, axis=0, keepdims=True)


def _chamfer_body(pb_ref, gb_ref, nha_ref, nb_ref,
                  ip_ref, cn_ref, ci_ref):
    rb = pl.program_id(1)
    pb = pb_ref[0]
    gb = gb_ref[0]
    nha = nha_ref[0]
    nb = nb_ref[0]
    u = lax.dot_general(pb, gb, (((0,), (0,)), ((), ())),
                        preferred_element_type=jnp.float32)
    t1 = u + nb
    colidx = lax.broadcasted_iota(jnp.int32, (ROW_BLK, N_PAD), 1)
    rn = jnp.max(t1, axis=1, keepdims=True)
    ip_ref[0] = jnp.min(jnp.where(t1 == rn, colidx, BIG_I),
                        axis=1, keepdims=True)
    t2 = u + nha
    rowidx = (lax.broadcasted_iota(jnp.int32, (ROW_BLK, N_PAD), 0)
              + rb * ROW_BLK)
    cn = jnp.max(t2, axis=0, keepdims=True)
    cn_ref[0, 0] = cn
    ci_ref[0, 0] = jnp.min(jnp.where(t2 == cn, rowidx, BIG_I), axis=0,
                           keepdims=True)


def _select_body(cn_ref, ci_ref, ig_ref):
    cn = cn_ref[...]
    ci = ci_ref[...]
    cg = jnp.max(cn, axis=1, keepdims=True)
    ig_ref[...] = jnp.min(jnp.where(cn == cg, ci, BIG_I), axis=1)


def _sc_exact(px, py, pz, gx, gy, gz, ip, ig, dp_out, dg_out,
              pxb, pyb, pzb, gxb, gyb, gzb, ipc, igc, dpc, dgc):
    nc = 2
    w = lax.axis_index("s") * nc + lax.axis_index("c")
    b = w // 8
    c0 = (w % 8) * CHUNK

    pltpu.sync_copy(px.at[pl.ds(b * N_PAD, N_PAD)], pxb)
    pltpu.sync_copy(py.at[pl.ds(b * N_PAD, N_PAD)], pyb)
    pltpu.sync_copy(pz.at[pl.ds(b * N_PAD, N_PAD)], pzb)
    pltpu.sync_copy(gx.at[pl.ds(b * N_PAD, N_PAD)], gxb)
    pltpu.sync_copy(gy.at[pl.ds(b * N_PAD, N_PAD)], gyb)
    pltpu.sync_copy(gz.at[pl.ds(b * N_PAD, N_PAD)], gzb)
    pltpu.sync_copy(ip.at[pl.ds(w * CHUNK, CHUNK)], ipc)
    pltpu.sync_copy(ig.at[pl.ds(w * CHUNK, CHUNK)], igc)

    def body(i, _):
        s = pl.ds(i * 16, 16)
        sc = pl.ds(c0 + i * 16, 16)
        iv = ipc[s]
        nx = plsc.load_gather(gxb, [iv])
        ny = plsc.load_gather(gyb, [iv])
        nz = plsc.load_gather(gzb, [iv])
        dx = pxb[sc] - nx
        dy = pyb[sc] - ny
        dz = pzb[sc] - nz
        dpc[s] = dx * dx + dy * dy + dz * dz
        jv = igc[s]
        mx = plsc.load_gather(pxb, [jv])
        my = plsc.load_gather(pyb, [jv])
        mz = plsc.load_gather(pzb, [jv])
        ex = gxb[sc] - mx
        ey = gyb[sc] - my
        ez = gzb[sc] - mz
        dgc[s] = ex * ex + ey * ey + ez * ez
        return 0

    lax.fori_loop(0, CHUNK // 16, body, 0, unroll=2)

    pltpu.sync_copy(dpc, dp_out.at[pl.ds(w * CHUNK, CHUNK)])
    pltpu.sync_copy(dgc, dg_out.at[pl.ds(w * CHUNK, CHUNK)])


def _finalize_body(dp_ref, dg_ref, out_ref):
    def stats(d2):
        mask = lax.broadcasted_iota(jnp.int32, d2.shape, 1) < N_REAL
        d = jnp.sqrt(jnp.maximum(d2, 1e-12))
        mean_d = jnp.sum(jnp.where(mask, d, 0.0), axis=1) / N_REAL
        frac = jnp.sum(jnp.where(mask & (d < FS_T), 1.0, 0.0), axis=1) / N_REAL
        return mean_d, frac

    mp, fp = stats(dp_ref[...])
    mg, fg = stats(dg_ref[...])
    cd = (mp + mg) / 2.0
    fs = 2.0 * fp * fg / (fp + fg + 1e-08)
    out_ref[...] = jnp.stack([cd, fs], axis=0)


def kernel(pred, gt):
    B, N, _ = pred.shape

    def prep(x):
        x = jnp.pad(x, ((0, 0), (0, N_PAD - N), (0, 0)),
                    constant_values=PAD_COORD)
        return jnp.pad(x, ((0, 0), (0, 0), (0, 5)))

    p8 = prep(pred)
    g8 = prep(gt)
    pt = p8.transpose(0, 2, 1)
    gt8 = g8.transpose(0, 2, 1)
    pbt = pt.astype(jnp.bfloat16)
    gbt = gt8.astype(jnp.bfloat16)

    nha, nb = pl.pallas_call(
        _norms_body,
        grid=(B,),
        in_specs=[
            pl.BlockSpec((1, N_PAD, 8), lambda b: (b, 0, 0)),
            pl.BlockSpec((1, 8, N_PAD), lambda b: (b, 0, 0)),
        ],
        out_specs=[
            pl.BlockSpec((1, N_PAD, 1), lambda b: (b, 0, 0)),
            pl.BlockSpec((1, 1, N_PAD), lambda b: (b, 0, 0)),
        ],
        out_shape=[
            jax.ShapeDtypeStruct((B, N_PAD, 1), jnp.float32),
            jax.ShapeDtypeStruct((B, 1, N_PAD), jnp.float32),
        ],
    )(p8, gt8)

    colspec = pl.BlockSpec((1, 1, 1, N_PAD), lambda b, rb: (b, rb, 0, 0))

    ip, cn, ci = pl.pallas_call(
        _chamfer_body,
        grid=(B, RB),
        in_specs=[
            pl.BlockSpec((1, 8, ROW_BLK), lambda b, rb: (b, 0, rb)),
            pl.BlockSpec((1, 8, N_PAD), lambda b, rb: (b, 0, 0)),
            pl.BlockSpec((1, ROW_BLK, 1), lambda b, rb: (b, rb, 0)),
            pl.BlockSpec((1, 1, N_PAD), lambda b, rb: (b, 0, 0)),
        ],
        out_specs=[
            pl.BlockSpec((1, ROW_BLK, 1), lambda b, rb: (b, rb, 0)),
            colspec, colspec,
        ],
        out_shape=[
            jax.ShapeDtypeStruct((B, N_PAD, 1), jnp.int32),
            jax.ShapeDtypeStruct((B, RB, 1, N_PAD), jnp.float32),
            jax.ShapeDtypeStruct((B, RB, 1, N_PAD), jnp.int32),
        ],
    )(pbt, gbt, nha, nb)

    ig = pl.pallas_call(
        _select_body,
        out_shape=jax.ShapeDtypeStruct((B, N_PAD), jnp.int32),
    )(cn.reshape(B, RB, N_PAD), ci.reshape(B, RB, N_PAD))

    p3 = pt[:, :3, :].transpose(1, 0, 2).reshape(3, B * N_PAD)
    g3 = gt8[:, :3, :].transpose(1, 0, 2).reshape(3, B * N_PAD)

    mesh = plsc.VectorSubcoreMesh(core_axis_name="c", subcore_axis_name="s",
                                  num_cores=2, num_subcores=16)
    sc = functools.partial(
        pl.kernel,
        mesh=mesh,
        compiler_params=pltpu.CompilerParams(needs_layout_passes=False),
        out_type=[
            jax.ShapeDtypeStruct((B * N_PAD,), jnp.float32),
            jax.ShapeDtypeStruct((B * N_PAD,), jnp.float32),
        ],
        scratch_types=[
            pltpu.VMEM((N_PAD,), jnp.float32),
            pltpu.VMEM((N_PAD,), jnp.float32),
            pltpu.VMEM((N_PAD,), jnp.float32),
            pltpu.VMEM((N_PAD,), jnp.float32),
            pltpu.VMEM((N_PAD,), jnp.float32),
            pltpu.VMEM((N_PAD,), jnp.float32),
            pltpu.VMEM((CHUNK,), jnp.int32),
            pltpu.VMEM((CHUNK,), jnp.int32),
            pltpu.VMEM((CHUNK,), jnp.float32),
            pltpu.VMEM((CHUNK,), jnp.float32),
        ],
    )(_sc_exact)
    dp, dg = sc(p3[0], p3[1], p3[2], g3[0], g3[1], g3[2],
                ip.reshape(B * N_PAD), ig.reshape(B * N_PAD))

    out = pl.pallas_call(
        _finalize_body,
        out_shape=jax.ShapeDtypeStruct((2, B), jnp.float32),
    )(dp.reshape(B, N_PAD), dg.reshape(B, N_PAD))
    return out
